# baseline (speedup 1.0000x reference)
import functools

import jax
import jax.numpy as jnp
from jax import lax
from jax.experimental import pallas as pl
from jax.experimental.pallas import tpu as pltpu
from jax.experimental.pallas import tpu_sc as plsc

_NUM_CORES = 2
_NUM_SUBCORES = 16
_NUM_WORKERS = _NUM_CORES * _NUM_SUBCORES
_STAGE = 8192
_LANE = 128
_R_CHUNK = 64
_GROUP = 8


@functools.lru_cache(maxsize=None)
def _make_gather(batch, seq, vocab):
    assert batch % _NUM_WORKERS == 0
    r_per_w = batch // _NUM_WORKERS
    n_pieces = -(-vocab // _STAGE)
    tail = vocab - (n_pieces - 1) * _STAGE
    rounds = -(-n_pieces // _NUM_SUBCORES)
    assert r_per_w % _R_CHUNK == 0
    n_chunks = r_per_w // _R_CHUNK
    full, rem = divmod(seq, 16)
    _SPAD = (full + (1 if rem else 0)) * 16
    mesh = plsc.VectorSubcoreMesh(core_axis_name="c", subcore_axis_name="s")

    @functools.partial(
        pl.kernel,
        mesh=mesh,
        out_type=jax.ShapeDtypeStruct((batch, seq), jnp.int32),
        scratch_types=[
            pltpu.VMEM_SHARED((vocab,), jnp.int32),
            pltpu.VMEM((_STAGE,), jnp.int32),
            pltpu.VMEM((_R_CHUNK, seq), jnp.int32),
            pltpu.VMEM((_R_CHUNK * _SPAD,), jnp.int32),
            pltpu.VMEM((_R_CHUNK * _SPAD,), jnp.int32),
            pltpu.VMEM((_R_CHUNK, seq), jnp.int32),
            pltpu.SemaphoreType.DMA,
        ],
    )
    def k(vocab_hbm, tok_hbm, out_hbm, table_sh, stage_v, idx2_v, idx_v, rows_v, rows2_v, sem):
        sid = lax.axis_index("s")
        wid = sid * _NUM_CORES + lax.axis_index("c")
        row0 = wid * r_per_w
        for r in range(rounds):
            q = r * _NUM_SUBCORES + sid

            @pl.when(q < n_pieces - 1)
            def _():
                p = q * _STAGE
                pltpu.sync_copy(vocab_hbm.at[pl.ds(p, _STAGE)], stage_v)
                pltpu.sync_copy(stage_v, table_sh.at[pl.ds(p, _STAGE)])

            @pl.when(q == n_pieces - 1)
            def _():
                p = q * _STAGE
                pltpu.sync_copy(vocab_hbm.at[pl.ds(p, tail)], stage_v.at[pl.ds(0, tail)])
                pltpu.sync_copy(stage_v.at[pl.ds(0, tail)], table_sh.at[pl.ds(p, tail)])

        plsc.subcore_barrier()


        tails0 = _R_CHUNK * full * 16

        def row_compact(r):
            for j in range(full):
                idx_v[pl.ds((r * full + j) * 16, 16)] = idx2_v[r, pl.ds(j * 16, 16)]
            if rem:
                v = idx2_v[r, pl.ds(seq - 16, 16)]
                idx_v[pl.ds(tails0 + r * 16, 16)] = v

        def row_expand(r):
            if rem:
                w = rows_v[pl.ds(tails0 + r * 16, 16)]
                rows2_v[r, pl.ds(seq - 16, 16)] = w
            for j in range(full):
                rows2_v[r, pl.ds(j * 16, 16)] = rows_v[pl.ds((r * full + j) * 16, 16)]

        def chunk(ci, carry):
            r0 = row0 + ci * _R_CHUNK
            pltpu.sync_copy(tok_hbm.at[pl.ds(r0, _R_CHUNK)], idx2_v)

            def compact(r, carry2):
                row_compact(r)
                return carry2

            lax.fori_loop(0, _R_CHUNK, compact, 0)
            pltpu.async_copy(table_sh.at[idx_v], rows_v, sem).wait()

            def expand(r, carry2):
                row_expand(r)
                return carry2

            lax.fori_loop(0, _R_CHUNK, expand, 0)
            pltpu.sync_copy(rows2_v, out_hbm.at[pl.ds(r0, _R_CHUNK)])
            return carry

        lax.fori_loop(0, n_chunks, chunk, 0)

    return k


def kernel(tokens, vocab_map):
    batch, seq = tokens.shape
    return _make_gather(batch, seq, vocab_map.shape[0])(vocab_map, tokens)


# speedup vs baseline: 1.1214x; 1.1214x over previous
import functools

import jax
import jax.numpy as jnp
from jax import lax
from jax.experimental import pallas as pl
from jax.experimental.pallas import tpu as pltpu
from jax.experimental.pallas import tpu_sc as plsc

_NUM_CORES = 2
_NUM_SUBCORES = 16
_NUM_WORKERS = _NUM_CORES * _NUM_SUBCORES
_STAGE = 8192
_R_CHUNK = 32


@functools.lru_cache(maxsize=None)
def _make_gather(batch, seq, vocab):
    assert batch % _NUM_WORKERS == 0
    r_per_w = batch // _NUM_WORKERS
    n_pieces = -(-vocab // _STAGE)
    tail = vocab - (n_pieces - 1) * _STAGE
    rounds = -(-n_pieces // _NUM_SUBCORES)
    assert r_per_w % _R_CHUNK == 0
    n_chunks = r_per_w // _R_CHUNK
    full, rem = divmod(seq, 16)
    flat_words = _R_CHUNK * (full + (1 if rem else 0)) * 16
    tails0 = _R_CHUNK * full * 16
    mesh = plsc.VectorSubcoreMesh(core_axis_name="c", subcore_axis_name="s")

    @functools.partial(
        pl.kernel,
        mesh=mesh,
        out_type=jax.ShapeDtypeStruct((batch, seq), jnp.int32),
        scratch_types=[
            pltpu.VMEM_SHARED((vocab,), jnp.int32),
            pltpu.VMEM((_STAGE,), jnp.int32),
            pltpu.VMEM((_R_CHUNK, seq), jnp.int32),
            pltpu.VMEM((_R_CHUNK, seq), jnp.int32),
            pltpu.VMEM((flat_words,), jnp.int32),
            pltpu.VMEM((flat_words,), jnp.int32),
            pltpu.VMEM((flat_words,), jnp.int32),
            pltpu.VMEM((flat_words,), jnp.int32),
            pltpu.VMEM((_R_CHUNK, seq), jnp.int32),
            pltpu.VMEM((_R_CHUNK, seq), jnp.int32),
            pltpu.SemaphoreType.DMA,
            pltpu.SemaphoreType.DMA,
            pltpu.SemaphoreType.DMA,
            pltpu.SemaphoreType.DMA,
            pltpu.SemaphoreType.DMA,
        ],
    )
    def k(
        vocab_hbm,
        tok_hbm,
        out_hbm,
        table_sh,
        stage_v,
        idx2_a,
        idx2_b,
        fidx_a,
        fidx_b,
        fres_a,
        fres_b,
        rows2_a,
        rows2_b,
        sem_in_a,
        sem_in_b,
        sem_g,
        sem_out_a,
        sem_out_b,
    ):
        sid = lax.axis_index("s")
        wid = sid * _NUM_CORES + lax.axis_index("c")
        row0 = wid * r_per_w
        idx2 = (idx2_a, idx2_b)
        fidx = (fidx_a, fidx_b)
        fres = (fres_a, fres_b)
        rows2 = (rows2_a, rows2_b)
        sem_in = (sem_in_a, sem_in_b)
        sem_out = (sem_out_a, sem_out_b)

        def copy_in(c):
            return pltpu.async_copy(
                tok_hbm.at[pl.ds(row0 + c * _R_CHUNK, _R_CHUNK)],
                idx2[c % 2],
                sem_in[c % 2],
            )

        def copy_out(c):
            return pltpu.async_copy(
                rows2[c % 2],
                out_hbm.at[pl.ds(row0 + c * _R_CHUNK, _R_CHUNK)],
                sem_out[c % 2],
            )

        in_cp = {0: copy_in(0), 1: copy_in(1)}

        for r in range(rounds):
            q = r * _NUM_SUBCORES + sid

            @pl.when(q < n_pieces - 1)
            def _():
                p = q * _STAGE
                pltpu.sync_copy(vocab_hbm.at[pl.ds(p, _STAGE)], stage_v)
                pltpu.sync_copy(stage_v, table_sh.at[pl.ds(p, _STAGE)])

            @pl.when(q == n_pieces - 1)
            def _():
                p = q * _STAGE
                pltpu.sync_copy(vocab_hbm.at[pl.ds(p, tail)], stage_v.at[pl.ds(0, tail)])
                pltpu.sync_copy(stage_v.at[pl.ds(0, tail)], table_sh.at[pl.ds(p, tail)])


        def compact(c):
            src2, dst1 = idx2[c % 2], fidx[c % 2]

            def body(r, carry):
                for j in range(full):
                    dst1[pl.ds((r * full + j) * 16, 16)] = src2[r, pl.ds(j * 16, 16)]
                if rem:
                    v = src2[r, pl.ds(seq - 16, 16)]
                    dst1[pl.ds(tails0 + r * 16, 16)] = v
                return carry

            lax.fori_loop(0, _R_CHUNK, body, 0)

        def expand(c):
            src1, dst2 = fres[c % 2], rows2[c % 2]

            def body(r, carry):
                if rem:
                    w = src1[pl.ds(tails0 + r * 16, 16)]
                    dst2[r, pl.ds(seq - 16, 16)] = w
                for j in range(full):
                    dst2[r, pl.ds(j * 16, 16)] = src1[pl.ds((r * full + j) * 16, 16)]
                return carry

            lax.fori_loop(0, _R_CHUNK, body, 0)

        in_cp[0].wait()
        compact(0)
        plsc.subcore_barrier()

        out_cp = {}
        for c in range(n_chunks):
            g_cp = pltpu.async_copy(table_sh.at[fidx[c % 2]], fres[c % 2], sem_g)
            if c + 1 < n_chunks:
                if c + 2 < n_chunks:
                    in_cp[c + 2] = copy_in(c + 2)
                in_cp[c + 1].wait()
                compact(c + 1)
            g_cp.wait()
            if c >= 2:
                out_cp[c - 2].wait()
            expand(c)
            out_cp[c] = copy_out(c)
        for c in (n_chunks - 2, n_chunks - 1):
            if c >= 0:
                out_cp[c].wait()

    return k


def kernel(tokens, vocab_map):
    batch, seq = tokens.shape
    return _make_gather(batch, seq, vocab_map.shape[0])(vocab_map, tokens)


# speedup vs baseline: 1.3778x; 1.2286x over previous
import functools

import jax
import jax.numpy as jnp
from jax import lax
from jax.experimental import pallas as pl
from jax.experimental.pallas import tpu as pltpu
from jax.experimental.pallas import tpu_sc as plsc

_NUM_CORES = 2
_NUM_SUBCORES = 16
_NUM_WORKERS = _NUM_CORES * _NUM_SUBCORES
_STAGE = 8192


@functools.lru_cache(maxsize=None)
def _make_gather(n, vocab):
    assert n % (8 * _NUM_WORKERS) == 0
    b_per_w = n // _NUM_WORKERS
    n_pieces = -(-vocab // _STAGE)
    tail = vocab - (n_pieces - 1) * _STAGE
    rounds = -(-n_pieces // _NUM_SUBCORES)
    mesh = plsc.VectorSubcoreMesh(core_axis_name="c", subcore_axis_name="s")

    @functools.partial(
        pl.kernel,
        mesh=mesh,
        out_type=jax.ShapeDtypeStruct((n,), jnp.int32),
        scratch_types=[
            pltpu.VMEM_SHARED((vocab,), jnp.int32),
            pltpu.VMEM((_STAGE,), jnp.int32),
            pltpu.VMEM((b_per_w,), jnp.int32),
            pltpu.VMEM((b_per_w,), jnp.int32),
            pltpu.SemaphoreType.DMA,
            pltpu.SemaphoreType.DMA,
        ],
    )
    def k(vocab_hbm, tok_hbm, out_hbm, table_sh, stage_v, idx_v, rows_v, sem, sem_in):
        sid = lax.axis_index("s")
        wid = sid * _NUM_CORES + lax.axis_index("c")
        base = wid * b_per_w
        in_cp = pltpu.async_copy(tok_hbm.at[pl.ds(base, b_per_w)], idx_v, sem_in)

        for r in range(rounds):
            q = r * _NUM_SUBCORES + sid

            @pl.when(q < n_pieces - 1)
            def _():
                p = q * _STAGE
                pltpu.sync_copy(vocab_hbm.at[pl.ds(p, _STAGE)], stage_v)
                pltpu.sync_copy(stage_v, table_sh.at[pl.ds(p, _STAGE)])

            @pl.when(q == n_pieces - 1)
            def _():
                p = q * _STAGE
                pltpu.sync_copy(vocab_hbm.at[pl.ds(p, tail)], stage_v.at[pl.ds(0, tail)])
                pltpu.sync_copy(stage_v.at[pl.ds(0, tail)], table_sh.at[pl.ds(p, tail)])

        in_cp.wait()
        plsc.subcore_barrier()
        pltpu.async_copy(table_sh.at[idx_v], rows_v, sem).wait()
        pltpu.sync_copy(rows_v, out_hbm.at[pl.ds(base, b_per_w)])

    return k


def kernel(tokens, vocab_map):
    batch, seq = tokens.shape
    flat = tokens.T.reshape(-1)
    out = _make_gather(flat.shape[0], vocab_map.shape[0])(vocab_map, flat)
    return out.reshape(seq, batch).T
